# baseline (speedup 1.0000x reference)
import jax
import jax.numpy as jnp
from jax import lax
from jax.experimental import pallas as pl
from jax.experimental.pallas import tpu as pltpu
from jax.experimental.pallas import tpu_sc as plsc

_N = 262144
_D = 128
_B = 4096
_CLAMP_LO = 32
_CLAMP_HI = 96

_MROWS = 8

_CBLK = 16384
_NPREP = _N // _CBLK
_MBLK = _B // _NPREP


def _clamp_prep_body(x_ref, c_row, p_row, trig, o_ref, add_out):
    i = pl.program_id(0)

    xv = x_ref[...]
    col = lax.broadcasted_iota(jnp.int32, xv.shape, 1)
    mid = (col >= _CLAMP_LO) & (col < _CLAMP_HI)
    o_ref[...] = jnp.where(mid, jnp.minimum(xv, 1.0), xv)

    b = c_row[...] + p_row[...]
    r, h = i // 2, (i % 2) * _MBLK
    a_row = (c_row[pl.ds(r, 1), pl.ds(h, _MBLK)]
             + p_row[pl.ds(r, 1), pl.ds(h, _MBLK)])
    a = jnp.transpose(a_row, (1, 0))
    acc = jnp.zeros((_MBLK, _B // _MROWS), dtype=jnp.int32)
    for k in range(_MROWS):
        acc = acc + (a == b[k : k + 1, :]).astype(jnp.int32)
    m = jnp.sum(acc, axis=1, keepdims=True)
    add_out[...] = m.astype(jnp.float32) * trig[...]


_full = lambda i: (0, 0)
_clamp_prep_call = pl.pallas_call(
    _clamp_prep_body,
    grid=(_N // _CBLK,),
    in_specs=[
        pl.BlockSpec((_CBLK, _D), lambda i: (i, 0)),
        pl.BlockSpec((_MROWS, _B // _MROWS), _full),
        pl.BlockSpec((_MROWS, _B // _MROWS), _full),
        pl.BlockSpec((1, _D), _full),
    ],
    out_specs=[
        pl.BlockSpec((_CBLK, _D), lambda i: (i, 0)),
        pl.BlockSpec((_MBLK, _D), lambda i: (i, 0)),
    ],
    out_shape=[
        jax.ShapeDtypeStruct((_N, _D), jnp.float32),
        jax.ShapeDtypeStruct((_B, _D), jnp.float32),
    ],
)

_NC = 2
_NS = 16
_NW = _NC * _NS
_RPW = _B // _NW
_L = 16


def _sc_scatter_body(
    x_hbm, ce_hbm, pt_hbm, add_hbm, out_ref, idx_v, tmp_v, rows_v, add_v, sem
):
    wid = lax.axis_index("s") * _NC + lax.axis_index("c")
    base = wid * _RPW
    pltpu.sync_copy(ce_hbm.at[pl.ds(base, _RPW)], idx_v)
    pltpu.sync_copy(pt_hbm.at[pl.ds(base, _RPW)], tmp_v)
    for k in range(_RPW // _L):
        s = pl.ds(k * _L, _L)
        idx_v[s] = idx_v[s] + tmp_v[s]
    pltpu.sync_copy(add_hbm.at[pl.ds(base, _RPW)], add_v)
    pltpu.async_copy(x_hbm.at[idx_v], rows_v, sem).wait()

    @pl.loop(0, _RPW)
    def _row(j):
        for c in range(_D // _L):
            v = rows_v[j, pl.ds(c * _L, _L)] + add_v[j, pl.ds(c * _L, _L)]
            if _CLAMP_LO <= c * _L < _CLAMP_HI:
                v = jnp.minimum(v, 1.0)
            rows_v[j, pl.ds(c * _L, _L)] = v

    pltpu.async_copy(rows_v, out_ref.at[idx_v], sem).wait()


_sc_scatter = pl.kernel(
    _sc_scatter_body,
    out_type=(),
    mesh=plsc.VectorSubcoreMesh(
        core_axis_name="c", subcore_axis_name="s", num_cores=_NC, num_subcores=_NS
    ),
    scratch_types=[
        pltpu.VMEM((_RPW,), jnp.int32),
        pltpu.VMEM((_RPW,), jnp.int32),
        pltpu.VMEM((_RPW, _D), jnp.float32),
        pltpu.VMEM((_RPW, _D), jnp.float32),
        pltpu.SemaphoreType.DMA,
    ],
)


def kernel(x, center, ptr, trigger):
    ptr_head = ptr[:_B]
    c_row = center.reshape(_MROWS, _B // _MROWS)
    p_row = ptr_head.reshape(_MROWS, _B // _MROWS)
    trig = trigger.reshape(1, _D)

    out1, addend = _clamp_prep_call(x, c_row, p_row, trig)

    ref = jax.new_ref(out1)
    _sc_scatter(x, center, ptr_head, addend, ref)
    return jax.freeze(ref)


# speedup vs baseline: 1.0007x; 1.0007x over previous
import jax
import jax.numpy as jnp
from jax import lax
from jax.experimental import pallas as pl
from jax.experimental.pallas import tpu as pltpu
from jax.experimental.pallas import tpu_sc as plsc

_N = 262144
_D = 128
_B = 4096
_CLAMP_LO = 32
_CLAMP_HI = 96

_MROWS = 8

_CBLK = 16384
_NPREP = _N // _CBLK
_MBLK = _B // _NPREP


def _clamp_prep_body(x_ref, c_row, p_row, trig, o_ref, add_out):
    i = pl.program_id(0)

    xv = x_ref[...]
    col = lax.broadcasted_iota(jnp.int32, xv.shape, 1)
    mid = (col >= _CLAMP_LO) & (col < _CLAMP_HI)
    o_ref[...] = jnp.where(mid, jnp.minimum(xv, 1.0), xv)

    b = c_row[...] + p_row[...]
    r, h = i // 2, (i % 2) * _MBLK
    a_row = (c_row[pl.ds(r, 1), pl.ds(h, _MBLK)]
             + p_row[pl.ds(r, 1), pl.ds(h, _MBLK)])
    a = jnp.transpose(a_row, (1, 0))
    acc = jnp.zeros((_MBLK, _B // _MROWS), dtype=jnp.int32)
    for k in range(_MROWS):
        acc = acc + (a == b[k : k + 1, :]).astype(jnp.int32)
    m = jnp.sum(acc, axis=1, keepdims=True)
    add_out[pl.ds(i * _MBLK, _MBLK), :] = (
        m.astype(jnp.float32) * trig[...])


_full = lambda i: (0, 0)
_clamp_prep_call = pl.pallas_call(
    _clamp_prep_body,
    grid=(_N // _CBLK,),
    in_specs=[
        pl.BlockSpec((_CBLK, _D), lambda i: (i, 0)),
        pl.BlockSpec((_MROWS, _B // _MROWS), _full),
        pl.BlockSpec((_MROWS, _B // _MROWS), _full),
        pl.BlockSpec((1, _D), _full),
    ],
    out_specs=[
        pl.BlockSpec((_CBLK, _D), lambda i: (i, 0)),
        pl.BlockSpec((_B, _D), _full),
    ],
    out_shape=[
        jax.ShapeDtypeStruct((_N, _D), jnp.float32),
        jax.ShapeDtypeStruct((_B, _D), jnp.float32),
    ],
)

_NC = 2
_NS = 16
_NW = _NC * _NS
_RPW = _B // _NW
_L = 16


def _sc_scatter_body(
    x_hbm, ce_hbm, pt_hbm, add_hbm, out_ref, idx_v, tmp_v, rows_v, add_v, sem
):
    wid = lax.axis_index("s") * _NC + lax.axis_index("c")
    base = wid * _RPW
    pltpu.sync_copy(ce_hbm.at[pl.ds(base, _RPW)], idx_v)
    pltpu.sync_copy(pt_hbm.at[pl.ds(base, _RPW)], tmp_v)
    for k in range(_RPW // _L):
        s = pl.ds(k * _L, _L)
        idx_v[s] = idx_v[s] + tmp_v[s]
    pltpu.sync_copy(add_hbm.at[pl.ds(base, _RPW)], add_v)
    pltpu.async_copy(x_hbm.at[idx_v], rows_v, sem).wait()

    @pl.loop(0, _RPW)
    def _row(j):
        for c in range(_D // _L):
            v = rows_v[j, pl.ds(c * _L, _L)] + add_v[j, pl.ds(c * _L, _L)]
            if _CLAMP_LO <= c * _L < _CLAMP_HI:
                v = jnp.minimum(v, 1.0)
            rows_v[j, pl.ds(c * _L, _L)] = v

    pltpu.async_copy(rows_v, out_ref.at[idx_v], sem).wait()


_sc_scatter = pl.kernel(
    _sc_scatter_body,
    out_type=(),
    mesh=plsc.VectorSubcoreMesh(
        core_axis_name="c", subcore_axis_name="s", num_cores=_NC, num_subcores=_NS
    ),
    scratch_types=[
        pltpu.VMEM((_RPW,), jnp.int32),
        pltpu.VMEM((_RPW,), jnp.int32),
        pltpu.VMEM((_RPW, _D), jnp.float32),
        pltpu.VMEM((_RPW, _D), jnp.float32),
        pltpu.SemaphoreType.DMA,
    ],
)


def kernel(x, center, ptr, trigger):
    ptr_head = ptr[:_B]
    c_row = center.reshape(_MROWS, _B // _MROWS)
    p_row = ptr_head.reshape(_MROWS, _B // _MROWS)
    trig = trigger.reshape(1, _D)

    out1, addend = _clamp_prep_call(x, c_row, p_row, trig)

    ref = jax.new_ref(out1)
    _sc_scatter(x, center, ptr_head, addend, ref)
    return jax.freeze(ref)


# speedup vs baseline: 1.0165x; 1.0158x over previous
import jax
import jax.numpy as jnp
from jax import lax
from jax.experimental import pallas as pl
from jax.experimental.pallas import tpu as pltpu
from jax.experimental.pallas import tpu_sc as plsc

_N = 262144
_D = 128
_B = 4096
_CLAMP_LO = 32
_CLAMP_HI = 96

_MROWS = 8

_CBLK = 16384
_NPREP = _N // _CBLK
_MBLK = _B // _NPREP


def _clamp_prep_body(x_ref, c_row, p_row, trig, o_ref, add_out):
    i = pl.program_id(0)

    xv = x_ref[...]
    col = lax.broadcasted_iota(jnp.int32, xv.shape, 1)
    mid = (col >= _CLAMP_LO) & (col < _CLAMP_HI)
    o_ref[...] = jnp.where(mid, jnp.minimum(xv, 1.0), xv)

    b = c_row[...] + p_row[...]
    r, h = i // 2, (i % 2) * _MBLK
    a_row = (c_row[pl.ds(r, 1), pl.ds(h, _MBLK)]
             + p_row[pl.ds(r, 1), pl.ds(h, _MBLK)])
    a = jnp.transpose(a_row, (1, 0))
    acc = jnp.zeros((_MBLK, _B // _MROWS), dtype=jnp.int32)
    for k in range(_MROWS):
        acc = acc + (a == b[k : k + 1, :]).astype(jnp.int32)
    m = jnp.sum(acc, axis=1, keepdims=True)
    add_out[pl.ds(i * _MBLK, _MBLK), :] = (
        m.astype(jnp.float32) * trig[...])


_full = lambda i: (0, 0)
_clamp_prep_call = pl.pallas_call(
    _clamp_prep_body,
    grid=(_N // _CBLK,),
    in_specs=[
        pl.BlockSpec((_CBLK, _D), lambda i: (i, 0)),
        pl.BlockSpec((_MROWS, _B // _MROWS), _full),
        pl.BlockSpec((_MROWS, _B // _MROWS), _full),
        pl.BlockSpec((1, _D), _full),
    ],
    out_specs=[
        pl.BlockSpec((_CBLK, _D), lambda i: (i, 0)),
        pl.BlockSpec((_B, _D), _full),
    ],
    out_shape=[
        jax.ShapeDtypeStruct((_N, _D), jnp.float32),
        jax.ShapeDtypeStruct((_B, _D), jnp.float32),
    ],
)

_NC = 2
_NS = 16
_NW = _NC * _NS
_RPW = _B // _NW
_L = 16


def _sc_scatter_body(
    x_hbm, ce_hbm, pt_hbm, add_hbm, out_ref, idx_v, tmp_v, rows_v, add_v, sem
):
    wid = lax.axis_index("s") * _NC + lax.axis_index("c")
    base = wid * _RPW
    c_in = pltpu.async_copy(ce_hbm.at[pl.ds(base, _RPW)], idx_v, sem)
    t_in = pltpu.async_copy(pt_hbm.at[pl.ds(base, _RPW)], tmp_v, sem)
    a_in = pltpu.async_copy(add_hbm.at[pl.ds(base, _RPW)], add_v, sem)
    c_in.wait()
    t_in.wait()
    for k in range(_RPW // _L):
        s = pl.ds(k * _L, _L)
        idx_v[s] = idx_v[s] + tmp_v[s]
    pltpu.async_copy(x_hbm.at[idx_v], rows_v, sem).wait()
    a_in.wait()

    @pl.loop(0, _RPW)
    def _row(j):
        for c in range(_D // _L):
            v = rows_v[j, pl.ds(c * _L, _L)] + add_v[j, pl.ds(c * _L, _L)]
            if _CLAMP_LO <= c * _L < _CLAMP_HI:
                v = jnp.minimum(v, 1.0)
            rows_v[j, pl.ds(c * _L, _L)] = v

    pltpu.async_copy(rows_v, out_ref.at[idx_v], sem).wait()


_sc_scatter = pl.kernel(
    _sc_scatter_body,
    out_type=(),
    mesh=plsc.VectorSubcoreMesh(
        core_axis_name="c", subcore_axis_name="s", num_cores=_NC, num_subcores=_NS
    ),
    scratch_types=[
        pltpu.VMEM((_RPW,), jnp.int32),
        pltpu.VMEM((_RPW,), jnp.int32),
        pltpu.VMEM((_RPW, _D), jnp.float32),
        pltpu.VMEM((_RPW, _D), jnp.float32),
        pltpu.SemaphoreType.DMA,
    ],
)


def kernel(x, center, ptr, trigger):
    ptr_head = ptr[:_B]
    c_row = center.reshape(_MROWS, _B // _MROWS)
    p_row = ptr_head.reshape(_MROWS, _B // _MROWS)
    trig = trigger.reshape(1, _D)

    out1, addend = _clamp_prep_call(x, c_row, p_row, trig)

    ref = jax.new_ref(out1)
    _sc_scatter(x, center, ptr_head, addend, ref)
    return jax.freeze(ref)


# speedup vs baseline: 1.0231x; 1.0065x over previous
import jax
import jax.numpy as jnp
from jax import lax
from jax.experimental import pallas as pl
from jax.experimental.pallas import tpu as pltpu
from jax.experimental.pallas import tpu_sc as plsc

_N = 262144
_D = 128
_B = 4096
_CLAMP_LO = 32
_CLAMP_HI = 96

_MROWS = 8

_CBLK = 16384
_NPREP = _N // _CBLK
_MBLK = _B // _NPREP


def _clamp_prep_body(x_ref, c_row, p_row, trig, o_ref, add_out):
    i = pl.program_id(0)

    xv = x_ref[...]
    col = lax.broadcasted_iota(jnp.int32, xv.shape, 1)
    mid = (col >= _CLAMP_LO) & (col < _CLAMP_HI)
    o_ref[...] = jnp.where(mid, jnp.minimum(xv, 1.0), xv)

    b = c_row[...] + p_row[...]
    r, h = i // 2, (i % 2) * _MBLK
    a_row = (c_row[pl.ds(r, 1), pl.ds(h, _MBLK)]
             + p_row[pl.ds(r, 1), pl.ds(h, _MBLK)])
    a = jnp.transpose(a_row, (1, 0))
    acc = jnp.zeros((_MBLK, _B // _MROWS), dtype=jnp.int32)
    for k in range(_MROWS):
        acc = acc + (a == b[k : k + 1, :]).astype(jnp.int32)
    m = jnp.sum(acc, axis=1, keepdims=True)
    add_out[pl.ds(i * _MBLK, _MBLK), :] = (
        m.astype(jnp.float32) * trig[...])


_full = lambda i: (0, 0)
_clamp_prep_call = pl.pallas_call(
    _clamp_prep_body,
    grid=(_N // _CBLK,),
    in_specs=[
        pl.BlockSpec((_CBLK, _D), lambda i: (i, 0)),
        pl.BlockSpec((_MROWS, _B // _MROWS), _full),
        pl.BlockSpec((_MROWS, _B // _MROWS), _full),
        pl.BlockSpec((1, _D), _full),
    ],
    out_specs=[
        pl.BlockSpec((_CBLK, _D), lambda i: (i, 0)),
        pl.BlockSpec((_B, _D), _full),
    ],
    out_shape=[
        jax.ShapeDtypeStruct((_N, _D), jnp.float32),
        jax.ShapeDtypeStruct((_B, _D), jnp.float32),
    ],
)

_NC = 2
_NS = 16
_NW = _NC * _NS
_RPW = _B // _NW
_L = 16


_RH = _RPW // 2


def _sc_scatter_body(
    x_hbm, ce_hbm, pt_hbm, add_hbm, out_ref, idx2_v, tmp_v, rows_v, add_v,
    semi, semg, sems
):
    wid = lax.axis_index("s") * _NC + lax.axis_index("c")
    base = wid * _RPW
    c0 = pltpu.async_copy(ce_hbm.at[pl.ds(base, _RH)], idx2_v.at[0], semi)
    c1 = pltpu.async_copy(ce_hbm.at[pl.ds(base + _RH, _RH)], idx2_v.at[1], semi)
    t_in = pltpu.async_copy(pt_hbm.at[pl.ds(base, _RPW)], tmp_v, semi)
    a_in = pltpu.async_copy(add_hbm.at[pl.ds(base, _RPW)], add_v, semi)
    c0.wait()
    c1.wait()
    t_in.wait()
    for h in range(2):
        for k in range(_RH // _L):
            s = pl.ds(k * _L, _L)
            idx2_v[h, s] = idx2_v[h, s] + tmp_v[pl.ds(h * _RH + k * _L, _L)]
    g0 = pltpu.async_copy(x_hbm.at[idx2_v.at[0]], rows_v.at[pl.ds(0, _RH)], semg)
    g1 = pltpu.async_copy(x_hbm.at[idx2_v.at[1]], rows_v.at[pl.ds(_RH, _RH)], semg)
    g0.wait()
    a_in.wait()

    def _compute(lo):
        @pl.loop(lo, lo + _RH)
        def _row(j):
            for c in range(_D // _L):
                v = rows_v[j, pl.ds(c * _L, _L)] + add_v[j, pl.ds(c * _L, _L)]
                if _CLAMP_LO <= c * _L < _CLAMP_HI:
                    v = jnp.minimum(v, 1.0)
                rows_v[j, pl.ds(c * _L, _L)] = v

    _compute(0)
    s0 = pltpu.async_copy(rows_v.at[pl.ds(0, _RH)], out_ref.at[idx2_v.at[0]], sems)
    g1.wait()
    _compute(_RH)
    s1 = pltpu.async_copy(rows_v.at[pl.ds(_RH, _RH)], out_ref.at[idx2_v.at[1]], sems)
    s0.wait()
    s1.wait()


_sc_scatter = pl.kernel(
    _sc_scatter_body,
    out_type=(),
    mesh=plsc.VectorSubcoreMesh(
        core_axis_name="c", subcore_axis_name="s", num_cores=_NC, num_subcores=_NS
    ),
    scratch_types=[
        pltpu.VMEM((2, _RPW // 2), jnp.int32),
        pltpu.VMEM((_RPW,), jnp.int32),
        pltpu.VMEM((_RPW, _D), jnp.float32),
        pltpu.VMEM((_RPW, _D), jnp.float32),
        pltpu.SemaphoreType.DMA,
        pltpu.SemaphoreType.DMA,
        pltpu.SemaphoreType.DMA,
    ],
)


def kernel(x, center, ptr, trigger):
    ptr_head = ptr[:_B]
    c_row = center.reshape(_MROWS, _B // _MROWS)
    p_row = ptr_head.reshape(_MROWS, _B // _MROWS)
    trig = trigger.reshape(1, _D)

    out1, addend = _clamp_prep_call(x, c_row, p_row, trig)

    ref = jax.new_ref(out1)
    _sc_scatter(x, center, ptr_head, addend, ref)
    return jax.freeze(ref)
